# Initial kernel scaffold; baseline (speedup 1.0000x reference)
import jax
import jax.numpy as jnp
from jax.experimental import pallas as pl


def kernel(x_node_feature, edge_index, node_type, W_content, b_content, W_agg, b_agg):
    raise NotImplementedError("write your pallas kernel here")


# speedup vs baseline: 8.7491x; 8.7491x over previous
import jax
import jax.numpy as jnp
from jax import lax
from jax.experimental import pallas as pl
from jax.experimental.pallas import tpu as pltpu
from jax.experimental.pallas import tpu_sc as plsc

N = 10000
E = 320000
D = 128
T = 3
NQ = 4
Q = D // NQ

NC = 2
NS = 16
K = 128
NCHUNK = E // K
MAXCH = 160
EPAD = MAXCH * K
SEGS = N * T
ROWS_PER_TILE = 2000

BN = 1000
GRID = N // BN


def _encode_body(x_ref, nt_ref, w_ref, b_ref, out_ref):
    x = x_ref[...]
    nt = nt_ref[...].reshape(BN, 1)
    acc = jnp.zeros((BN, D), jnp.float32)
    for t in range(T):
        e = lax.dot_general(x, w_ref[t], (((1,), (0,)), ((), ())),
                            preferred_element_type=jnp.float32)
        e = e + b_ref[t][None, :]
        acc = acc + jnp.where(nt == t, e, 0.0)
    acc = jnp.where(acc >= 0.0, acc, 0.01 * acc)
    for q in range(NQ):
        out_ref[q] = acc[:, q * Q:(q + 1) * Q]


def _encode(x, node_type, W_content, b_content):
    return pl.pallas_call(
        _encode_body,
        grid=(GRID,),
        in_specs=[
            pl.BlockSpec((BN, D), lambda i: (i, 0)),
            pl.BlockSpec((1, 1, BN), lambda i: (i, 0, 0)),
            pl.BlockSpec((T, D, D), lambda i: (0, 0, 0)),
            pl.BlockSpec((T, D), lambda i: (0, 0)),
        ],
        out_specs=pl.BlockSpec((NQ, BN, Q), lambda i: (0, i, 0)),
        out_shape=jax.ShapeDtypeStruct((NQ, N, Q), jnp.float32),
    )(x, node_type.reshape(GRID, 1, BN), W_content, b_content)


def _segsum_body(enc_hbm, src_hbm, dst_hbm, nt_hbm, zrows_hbm, ones_hbm,
                 sums_out, cnt_out,
                 sums_sp, ntb, srcb, sidx, rows, sem):
    c = lax.axis_index("c")
    s = lax.axis_index("s")
    ch0 = s * MAXCH
    nch = jnp.minimum(NCHUNK - ch0, MAXCH)
    r0 = s * ROWS_PER_TILE

    pltpu.sync_copy(nt_hbm, ntb)

    pltpu.sync_copy(src_hbm.at[pl.ds(ch0, MAXCH)], srcb)
    pltpu.sync_copy(dst_hbm.at[pl.ds(ch0, MAXCH)], sidx)

    def prep_outer(k, _):
        def inner(j, _):
            sl = pl.ds(j * 16, 16)
            sv = srcb[k, sl]
            dv = sidx[k, sl]
            tv = plsc.load_gather(ntb, [sv])
            sidx[k, sl] = tv * N + dv
            srcb[k, sl] = sv + (2 * c) * N
            return 0
        return lax.fori_loop(0, K // 16, inner, 0)

    lax.fori_loop(0, MAXCH, prep_outer, 0)

    for p in range(2):
        q = 2 * c + p

        if p == 1:
            def adv_outer(k, _):
                def inner(j, _):
                    sl = pl.ds(j * 16, 16)
                    srcb[k, sl] = srcb[k, sl] + N
                    return 0
                return lax.fori_loop(0, K // 16, inner, 0)
            lax.fori_loop(0, MAXCH, adv_outer, 0)

        @pl.when(s < NS - 1)
        def _():
            pltpu.sync_copy(zrows_hbm, sums_sp.at[pl.ds(r0, ROWS_PER_TILE)])

        plsc.subcore_barrier()

        def chunk_body(k, _):
            @pl.when(k < nch)
            def _():
                pltpu.async_copy(enc_hbm.at[srcb.at[k]], rows, sem).wait()
                pltpu.sync_copy(rows, sums_sp.at[sidx.at[k]], add=True)
            return 0

        lax.fori_loop(0, MAXCH, chunk_body, 0)
        plsc.subcore_barrier()

        @pl.when(s < NS - 1)
        def _():
            pltpu.sync_copy(sums_sp.at[pl.ds(r0, ROWS_PER_TILE)],
                            sums_out.at[pl.ds(q * SEGS + r0, ROWS_PER_TILE)])

        plsc.subcore_barrier()

    pltpu.sync_copy(ones_hbm, rows)

    @pl.when(s < NS - 1)
    def _():
        pltpu.sync_copy(zrows_hbm, sums_sp.at[pl.ds(r0, ROWS_PER_TILE)])

    plsc.subcore_barrier()

    def cnt_body(k, _):
        @pl.when(jnp.logical_and(k < nch, lax.rem(ch0 + k, 2) == c))
        def _():
            pltpu.sync_copy(rows, sums_sp.at[sidx.at[k]], add=True)
        return 0

    lax.fori_loop(0, MAXCH, cnt_body, 0)
    plsc.subcore_barrier()

    @pl.when(s < NS - 1)
    def _():
        pltpu.sync_copy(sums_sp.at[pl.ds(r0, ROWS_PER_TILE)],
                        cnt_out.at[pl.ds(c * SEGS + r0, ROWS_PER_TILE)])


def _segsum(enc4, src, dst, node_type):
    npad = NS * EPAD - E
    srcp = jnp.concatenate([src, jnp.zeros((npad,), jnp.int32)]).reshape(NS * MAXCH, K)
    dstp = jnp.concatenate([dst, jnp.zeros((npad,), jnp.int32)]).reshape(NS * MAXCH, K)
    zrows = jnp.zeros((ROWS_PER_TILE, Q), jnp.float32)
    ones = jnp.ones((K, Q), jnp.float32)
    mesh = plsc.VectorSubcoreMesh(core_axis_name="c", subcore_axis_name="s",
                                  num_cores=NC, num_subcores=NS)
    f = pl.kernel(
        _segsum_body,
        out_type=(jax.ShapeDtypeStruct((NQ * SEGS, Q), jnp.float32),
                  jax.ShapeDtypeStruct((NC * SEGS, Q), jnp.float32)),
        mesh=mesh,
        compiler_params=pltpu.CompilerParams(needs_layout_passes=False,
                                             use_tc_tiling_on_sc=False),
        scratch_types=[
            pltpu.VMEM_SHARED((SEGS, Q), jnp.float32),
            pltpu.VMEM((N,), jnp.int32),
            pltpu.VMEM((MAXCH, K), jnp.int32),
            pltpu.VMEM((MAXCH, K), jnp.int32),
            pltpu.VMEM((K, Q), jnp.float32),
            pltpu.SemaphoreType.DMA,
        ],
    )
    return f(enc4.reshape(NQ * N, Q), srcp, dstp, node_type, zrows, ones)


def _finish_body(sums_ref, cnt_ref, enc_ref, w_ref, b_ref, out_ref):
    i = pl.program_id(0)
    cnt = cnt_ref[0, :, 0, :, 0] + cnt_ref[1, :, 0, :, 0]
    pieces = []
    for t in range(T):
        denom = jnp.maximum(cnt[t], 1.0)[:, None]
        for q in range(NQ):
            pieces.append(sums_ref[q, t] / denom)
    for q in range(NQ):
        pieces.append(enc_ref[q])
    het = jnp.concatenate(pieces, axis=1)
    z = lax.dot_general(het, w_ref[...], (((1,), (0,)), ((), ())),
                        preferred_element_type=jnp.float32)
    z = z + b_ref[...]
    emb = 1.0 / (1.0 + jnp.exp(-z))
    part = jnp.sum(emb, axis=0, keepdims=True)

    @pl.when(i == 0)
    def _():
        out_ref[...] = jnp.zeros((1, D), jnp.float32)

    out_ref[...] += part

    @pl.when(i == GRID - 1)
    def _():
        out_ref[...] = out_ref[...] * (1.0 / N)


def _finish(sums, cnt, enc4, W_agg, b_agg):
    out = pl.pallas_call(
        _finish_body,
        grid=(GRID,),
        in_specs=[
            pl.BlockSpec((NQ, T, BN, Q), lambda i: (0, 0, i, 0)),
            pl.BlockSpec((NC, T, 1, BN, Q), lambda i: (0, 0, i, 0, 0)),
            pl.BlockSpec((NQ, BN, Q), lambda i: (0, i, 0)),
            pl.BlockSpec(((T + 1) * D, D), lambda i: (0, 0)),
            pl.BlockSpec((1, D), lambda i: (0, 0)),
        ],
        out_specs=pl.BlockSpec((1, D), lambda i: (0, 0)),
        out_shape=jax.ShapeDtypeStruct((1, D), jnp.float32),
    )(sums.reshape(NQ, T, N, Q), cnt.reshape(NC, T, GRID, BN, Q), enc4, W_agg,
      b_agg.reshape(1, D))
    return out.reshape(D)


def kernel(x_node_feature, edge_index, node_type, W_content, b_content, W_agg, b_agg):
    nt = node_type.astype(jnp.int32)
    src = edge_index[0].astype(jnp.int32)
    dst = edge_index[1].astype(jnp.int32)
    enc4 = _encode(x_node_feature, nt, W_content, b_content)
    sums, cnt = _segsum(enc4, src, dst, nt)
    return _finish(sums, cnt, enc4, W_agg, b_agg)


# speedup vs baseline: 9.1366x; 1.0443x over previous
import jax
import jax.numpy as jnp
from jax import lax
from jax.experimental import pallas as pl
from jax.experimental.pallas import tpu as pltpu
from jax.experimental.pallas import tpu_sc as plsc

N = 10000
E = 320000
D = 128
T = 3
NQ = 4
Q = D // NQ

NC = 2
NS = 16
K = 128
NCHUNK = E // K
MAXCH = 160
EPAD = MAXCH * K
SEGS = N * T
SEGS_PAD = SEGS + 16
ROWS_PER_TILE = 2000
NBUF = 4

BN = 1000
GRID = N // BN


def _encode_body(x_ref, nt_ref, w_ref, b_ref, out_ref):
    x = x_ref[...]
    nt = nt_ref[...].reshape(BN, 1)
    acc = jnp.zeros((BN, D), jnp.float32)
    for t in range(T):
        e = lax.dot_general(x, w_ref[t], (((1,), (0,)), ((), ())),
                            preferred_element_type=jnp.float32)
        e = e + b_ref[t][None, :]
        acc = acc + jnp.where(nt == t, e, 0.0)
    acc = jnp.where(acc >= 0.0, acc, 0.01 * acc)
    for q in range(NQ):
        out_ref[q] = acc[:, q * Q:(q + 1) * Q]


def _encode(x, node_type, W_content, b_content):
    return pl.pallas_call(
        _encode_body,
        grid=(GRID,),
        in_specs=[
            pl.BlockSpec((BN, D), lambda i: (i, 0)),
            pl.BlockSpec((1, 1, BN), lambda i: (i, 0, 0)),
            pl.BlockSpec((T, D, D), lambda i: (0, 0, 0)),
            pl.BlockSpec((T, D), lambda i: (0, 0)),
        ],
        out_specs=pl.BlockSpec((NQ, BN, Q), lambda i: (0, i, 0)),
        out_shape=jax.ShapeDtypeStruct((NQ, N, Q), jnp.float32),
    )(x, node_type.reshape(GRID, 1, BN), W_content, b_content)


def _segsum_body(enc_hbm, src_hbm, dst_hbm, nt_hbm, zrows_hbm, ones_hbm,
                 sums_out, cnt_out,
                 sums_sp, ntb, srcb, sidx,
                 rows0, rows1, rows2, rows3,
                 gs0, gs1, gs2, gs3, ss0, ss1, ss2, ss3):
    rows = (rows0, rows1, rows2, rows3)
    gsem = (gs0, gs1, gs2, gs3)
    ssem = (ss0, ss1, ss2, ss3)
    c = lax.axis_index("c")
    s = lax.axis_index("s")
    ch0 = s * MAXCH
    nch = jnp.minimum(NCHUNK - ch0, MAXCH)
    r0 = s * ROWS_PER_TILE

    pltpu.sync_copy(nt_hbm, ntb)

    pltpu.sync_copy(src_hbm.at[pl.ds(ch0, MAXCH)], srcb)
    pltpu.sync_copy(dst_hbm.at[pl.ds(ch0, MAXCH)], sidx)
    dummy = SEGS + lax.iota(jnp.int32, 16)

    def prep_outer(k, _):
        real = k < nch

        def inner(j, _):
            sl = pl.ds(j * 16, 16)
            sv = srcb[k, sl]
            dv = sidx[k, sl]
            tv = plsc.load_gather(ntb, [sv])
            sidx[k, sl] = jnp.where(real, tv * N + dv, dummy)
            srcb[k, sl] = sv + (2 * c) * N
            return 0
        return lax.fori_loop(0, K // 16, inner, 0)

    lax.fori_loop(0, MAXCH, prep_outer, 0)

    for p in range(2):
        q = 2 * c + p

        if p == 1:
            def adv_outer(k, _):
                def inner(j, _):
                    sl = pl.ds(j * 16, 16)
                    srcb[k, sl] = srcb[k, sl] + N
                    return 0
                return lax.fori_loop(0, K // 16, inner, 0)
            lax.fori_loop(0, MAXCH, adv_outer, 0)

        @pl.when(s < NS - 1)
        def _():
            pltpu.sync_copy(zrows_hbm, sums_sp.at[pl.ds(r0, ROWS_PER_TILE)])

        plsc.subcore_barrier()

        for b in range(NBUF):
            pltpu.async_copy(enc_hbm.at[srcb.at[b]], rows[b], gsem[b])

        def round_body(m, _):
            k = m * NBUF
            for b in range(NBUF):
                pltpu.make_async_copy(enc_hbm.at[srcb.at[k + b]],
                                      rows[b], gsem[b]).wait()
                pltpu.async_copy(rows[b], sums_sp.at[sidx.at[k + b]], ssem[b],
                                 add=True)
            for b in range(NBUF):
                pltpu.make_async_copy(rows[b], sums_sp.at[sidx.at[k + b]],
                                      ssem[b]).wait()

                @pl.when(k + NBUF + b < MAXCH)
                def _():
                    pltpu.async_copy(enc_hbm.at[srcb.at[k + NBUF + b]],
                                     rows[b], gsem[b])
            return 0

        lax.fori_loop(0, MAXCH // NBUF, round_body, 0)
        plsc.subcore_barrier()

        @pl.when(s < NS - 1)
        def _():
            pltpu.sync_copy(sums_sp.at[pl.ds(r0, ROWS_PER_TILE)],
                            sums_out.at[pl.ds(q * SEGS + r0, ROWS_PER_TILE)])

        plsc.subcore_barrier()

    pltpu.sync_copy(ones_hbm, rows0)

    @pl.when(s < NS - 1)
    def _():
        pltpu.sync_copy(zrows_hbm, sums_sp.at[pl.ds(r0, ROWS_PER_TILE)])

    plsc.subcore_barrier()

    def cnt_round(m, _):
        k = m * 2 * NBUF + c
        for b in range(NBUF):
            pltpu.async_copy(rows0, sums_sp.at[sidx.at[k + 2 * b]], ssem[b],
                             add=True)
        for b in range(NBUF):
            pltpu.make_async_copy(rows0, sums_sp.at[sidx.at[k + 2 * b]],
                                  ssem[b]).wait()
        return 0

    lax.fori_loop(0, MAXCH // (2 * NBUF), cnt_round, 0)
    plsc.subcore_barrier()

    @pl.when(s < NS - 1)
    def _():
        pltpu.sync_copy(sums_sp.at[pl.ds(r0, ROWS_PER_TILE)],
                        cnt_out.at[pl.ds(c * SEGS + r0, ROWS_PER_TILE)])


def _segsum(enc4, src, dst, node_type):
    npad = NS * EPAD - E
    srcp = jnp.concatenate([src, jnp.zeros((npad,), jnp.int32)]).reshape(NS * MAXCH, K)
    dstp = jnp.concatenate([dst, jnp.zeros((npad,), jnp.int32)]).reshape(NS * MAXCH, K)
    zrows = jnp.zeros((ROWS_PER_TILE, Q), jnp.float32)
    ones = jnp.ones((K, Q), jnp.float32)
    mesh = plsc.VectorSubcoreMesh(core_axis_name="c", subcore_axis_name="s",
                                  num_cores=NC, num_subcores=NS)
    f = pl.kernel(
        _segsum_body,
        out_type=(jax.ShapeDtypeStruct((NQ * SEGS, Q), jnp.float32),
                  jax.ShapeDtypeStruct((NC * SEGS, Q), jnp.float32)),
        mesh=mesh,
        compiler_params=pltpu.CompilerParams(needs_layout_passes=False,
                                             use_tc_tiling_on_sc=False),
        scratch_types=(
            [pltpu.VMEM_SHARED((SEGS_PAD, Q), jnp.float32),
             pltpu.VMEM((N,), jnp.int32),
             pltpu.VMEM((MAXCH, K), jnp.int32),
             pltpu.VMEM((MAXCH, K), jnp.int32)]
            + [pltpu.VMEM((K, Q), jnp.float32)] * NBUF
            + [pltpu.SemaphoreType.DMA] * (2 * NBUF)
        ),
    )
    return f(enc4.reshape(NQ * N, Q), srcp, dstp, node_type, zrows, ones)


def _finish_body(sums_ref, cnt_ref, enc_ref, w_ref, b_ref, out_ref):
    i = pl.program_id(0)
    cnt = cnt_ref[0, :, 0, :, 0] + cnt_ref[1, :, 0, :, 0]
    pieces = []
    for t in range(T):
        denom = jnp.maximum(cnt[t], 1.0)[:, None]
        for q in range(NQ):
            pieces.append(sums_ref[q, t] / denom)
    for q in range(NQ):
        pieces.append(enc_ref[q])
    het = jnp.concatenate(pieces, axis=1)
    z = lax.dot_general(het, w_ref[...], (((1,), (0,)), ((), ())),
                        preferred_element_type=jnp.float32)
    z = z + b_ref[...]
    emb = 1.0 / (1.0 + jnp.exp(-z))
    part = jnp.sum(emb, axis=0, keepdims=True)

    @pl.when(i == 0)
    def _():
        out_ref[...] = jnp.zeros((1, D), jnp.float32)

    out_ref[...] += part

    @pl.when(i == GRID - 1)
    def _():
        out_ref[...] = out_ref[...] * (1.0 / N)


def _finish(sums, cnt, enc4, W_agg, b_agg):
    out = pl.pallas_call(
        _finish_body,
        grid=(GRID,),
        in_specs=[
            pl.BlockSpec((NQ, T, BN, Q), lambda i: (0, 0, i, 0)),
            pl.BlockSpec((NC, T, 1, BN, Q), lambda i: (0, 0, i, 0, 0)),
            pl.BlockSpec((NQ, BN, Q), lambda i: (0, i, 0)),
            pl.BlockSpec(((T + 1) * D, D), lambda i: (0, 0)),
            pl.BlockSpec((1, D), lambda i: (0, 0)),
        ],
        out_specs=pl.BlockSpec((1, D), lambda i: (0, 0)),
        out_shape=jax.ShapeDtypeStruct((1, D), jnp.float32),
    )(sums.reshape(NQ, T, N, Q), cnt.reshape(NC, T, GRID, BN, Q), enc4, W_agg,
      b_agg.reshape(1, D))
    return out.reshape(D)


def kernel(x_node_feature, edge_index, node_type, W_content, b_content, W_agg, b_agg):
    nt = node_type.astype(jnp.int32)
    src = edge_index[0].astype(jnp.int32)
    dst = edge_index[1].astype(jnp.int32)
    enc4 = _encode(x_node_feature, nt, W_content, b_content)
    sums, cnt = _segsum(enc4, src, dst, nt)
    return _finish(sums, cnt, enc4, W_agg, b_agg)


# speedup vs baseline: 13.8953x; 1.5208x over previous
import jax
import jax.numpy as jnp
from jax import lax
from jax.experimental import pallas as pl
from jax.experimental.pallas import tpu as pltpu
from jax.experimental.pallas import tpu_sc as plsc

N = 10000
E = 320000
D = 128
T = 3
NQ = 4
Q = D // NQ
H = D // 2

NC = 2
NS = 16
K = 128
NCHUNK = E // K
MAXCH = 160
NW = 2
WCH = MAXCH // NW
SEGS = N * T
SEGS_PAD = SEGS + 16
ROWS_PER_TILE = 2000
NBUF = 4
CW = 16

BN = 1000
GRID = N // BN


def _encode_body(x_ref, nt_ref, w_ref, b_ref, out_ref, outh_ref):
    x = x_ref[...]
    nt = nt_ref[...].reshape(BN, 1)
    acc = jnp.zeros((BN, D), jnp.float32)
    for t in range(T):
        e = lax.dot_general(x, w_ref[t], (((1,), (0,)), ((), ())),
                            preferred_element_type=jnp.float32)
        e = e + b_ref[t][None, :]
        acc = acc + jnp.where(nt == t, e, 0.0)
    acc = jnp.where(acc >= 0.0, acc, 0.01 * acc)
    for q in range(NQ):
        out_ref[q] = acc[:, q * Q:(q + 1) * Q]
    for h in range(2):
        outh_ref[h] = acc[:, h * H:(h + 1) * H].astype(jnp.bfloat16)


def _encode(x, node_type, W_content, b_content):
    return pl.pallas_call(
        _encode_body,
        grid=(GRID,),
        in_specs=[
            pl.BlockSpec((BN, D), lambda i: (i, 0)),
            pl.BlockSpec((1, 1, BN), lambda i: (i, 0, 0)),
            pl.BlockSpec((T, D, D), lambda i: (0, 0, 0)),
            pl.BlockSpec((T, D), lambda i: (0, 0)),
        ],
        out_specs=[pl.BlockSpec((NQ, BN, Q), lambda i: (0, i, 0)),
                   pl.BlockSpec((2, BN, H), lambda i: (0, i, 0))],
        out_shape=[jax.ShapeDtypeStruct((NQ, N, Q), jnp.float32),
                   jax.ShapeDtypeStruct((2, N, H), jnp.bfloat16)],
    )(x, node_type.reshape(GRID, 1, BN), W_content, b_content)


def _segsum_body(enc_hbm, src_hbm, dst_hbm, nt_hbm, zrows_hbm, zcnt_hbm, ones_hbm,
                 sums_out, cnt_out,
                 sums_sp, cnt_sp, ntb, srcb, sidx, ones,
                 rows0, rows1, rows2, rows3,
                 gs0, gs1, gs2, gs3, ss0, ss1, ss2, ss3, cs0, cs1, cs2, cs3):
    rows = (rows0, rows1, rows2, rows3)
    gsem = (gs0, gs1, gs2, gs3)
    ssem = (ss0, ss1, ss2, ss3)
    csem = (cs0, cs1, cs2, cs3)
    c = lax.axis_index("c")
    s = lax.axis_index("s")
    ch0 = s * MAXCH
    nch = jnp.minimum(NCHUNK - ch0, MAXCH)
    r0 = s * ROWS_PER_TILE

    pltpu.sync_copy(nt_hbm, ntb)
    pltpu.sync_copy(ones_hbm, ones)

    @pl.when(s < NS - 1)
    def _():
        pltpu.sync_copy(zrows_hbm, sums_sp.at[pl.ds(r0, ROWS_PER_TILE)])
        pltpu.sync_copy(zcnt_hbm, cnt_sp.at[pl.ds(r0, ROWS_PER_TILE)])

    dummy = SEGS + lax.iota(jnp.int32, 16)
    plsc.subcore_barrier()

    for w in range(NW):
        pltpu.sync_copy(src_hbm.at[pl.ds(ch0 + w * WCH, WCH)], srcb)
        pltpu.sync_copy(dst_hbm.at[pl.ds(ch0 + w * WCH, WCH)], sidx)

        def prep_outer(k, _):
            real = w * WCH + k < nch

            def inner(j, _):
                sl = pl.ds(j * 16, 16)
                sv = srcb[k, sl]
                dv = sidx[k, sl]
                tv = plsc.load_gather(ntb, [sv])
                sidx[k, sl] = jnp.where(real, tv * N + dv, dummy)
                srcb[k, sl] = sv + c * N
                return 0
            return lax.fori_loop(0, K // 16, inner, 0)

        lax.fori_loop(0, WCH, prep_outer, 0)

        for b in range(NBUF):
            pltpu.async_copy(enc_hbm.at[srcb.at[b]], rows[b], gsem[b])

        def round_body(m, _):
            k = m * NBUF
            for b in range(NBUF):
                pltpu.make_async_copy(enc_hbm.at[srcb.at[k + b]],
                                      rows[b], gsem[b]).wait()
                pltpu.async_copy(rows[b], sums_sp.at[sidx.at[k + b]], ssem[b],
                                 add=True)

                @pl.when(c == (b % 2))
                def _():
                    pltpu.async_copy(ones, cnt_sp.at[sidx.at[k + b]], csem[b],
                                     add=True)
            for b in range(NBUF):
                pltpu.make_async_copy(rows[b], sums_sp.at[sidx.at[k + b]],
                                      ssem[b]).wait()

                @pl.when(c == (b % 2))
                def _():
                    pltpu.make_async_copy(ones, cnt_sp.at[sidx.at[k + b]],
                                          csem[b]).wait()

                @pl.when(k + NBUF + b < WCH)
                def _():
                    pltpu.async_copy(enc_hbm.at[srcb.at[k + NBUF + b]],
                                     rows[b], gsem[b])
            return 0

        lax.fori_loop(0, WCH // NBUF, round_body, 0)

    plsc.subcore_barrier()

    @pl.when(s < NS - 1)
    def _():
        pltpu.sync_copy(sums_sp.at[pl.ds(r0, ROWS_PER_TILE)],
                        sums_out.at[pl.ds(c * SEGS + r0, ROWS_PER_TILE)])
        pltpu.sync_copy(cnt_sp.at[pl.ds(r0, ROWS_PER_TILE)],
                        cnt_out.at[pl.ds(c * SEGS + r0, ROWS_PER_TILE)])


def _segsum(ench, src, dst, node_type):
    npad = NS * MAXCH * K - E
    srcp = jnp.concatenate([src, jnp.zeros((npad,), jnp.int32)]).reshape(NS * MAXCH, K)
    dstp = jnp.concatenate([dst, jnp.zeros((npad,), jnp.int32)]).reshape(NS * MAXCH, K)
    zrows = jnp.zeros((ROWS_PER_TILE, H), jnp.bfloat16)
    zcnt = jnp.zeros((ROWS_PER_TILE, CW), jnp.bfloat16)
    ones = jnp.ones((K, CW), jnp.bfloat16)
    mesh = plsc.VectorSubcoreMesh(core_axis_name="c", subcore_axis_name="s",
                                  num_cores=NC, num_subcores=NS)
    f = pl.kernel(
        _segsum_body,
        out_type=(jax.ShapeDtypeStruct((NC * SEGS, H), jnp.bfloat16),
                  jax.ShapeDtypeStruct((NC * SEGS, CW), jnp.bfloat16)),
        mesh=mesh,
        compiler_params=pltpu.CompilerParams(needs_layout_passes=False,
                                             use_tc_tiling_on_sc=False),
        scratch_types=(
            [pltpu.VMEM_SHARED((SEGS_PAD, H), jnp.bfloat16),
             pltpu.VMEM_SHARED((SEGS_PAD, CW), jnp.bfloat16),
             pltpu.VMEM((N,), jnp.int32),
             pltpu.VMEM((WCH, K), jnp.int32),
             pltpu.VMEM((WCH, K), jnp.int32),
             pltpu.VMEM((K, CW), jnp.bfloat16)]
            + [pltpu.VMEM((K, H), jnp.bfloat16)] * NBUF
            + [pltpu.SemaphoreType.DMA] * (3 * NBUF)
        ),
    )
    return f(ench.reshape(2 * N, H), srcp, dstp, node_type, zrows, zcnt, ones)


def _finish_body(sums_ref, cnt_ref, enc_ref, w_ref, b_ref, out_ref):
    i = pl.program_id(0)
    cnt = (cnt_ref[0, :, 0, :, 0].astype(jnp.float32)
           + cnt_ref[1, :, 0, :, 0].astype(jnp.float32))
    pieces = []
    for t in range(T):
        denom = jnp.maximum(cnt[t], 1.0)[:, None]
        for h in range(2):
            pieces.append(sums_ref[h, t].astype(jnp.float32) / denom)
    for q in range(NQ):
        pieces.append(enc_ref[q])
    het = jnp.concatenate(pieces, axis=1)
    z = lax.dot_general(het, w_ref[...], (((1,), (0,)), ((), ())),
                        preferred_element_type=jnp.float32)
    z = z + b_ref[...]
    emb = 1.0 / (1.0 + jnp.exp(-z))
    part = jnp.sum(emb, axis=0, keepdims=True)

    @pl.when(i == 0)
    def _():
        out_ref[...] = jnp.zeros((1, D), jnp.float32)

    out_ref[...] += part

    @pl.when(i == GRID - 1)
    def _():
        out_ref[...] = out_ref[...] * (1.0 / N)


def _finish(sums, cnt, enc4, W_agg, b_agg):
    out = pl.pallas_call(
        _finish_body,
        grid=(GRID,),
        in_specs=[
            pl.BlockSpec((NC, T, BN, H), lambda i: (0, 0, i, 0)),
            pl.BlockSpec((NC, T, 1, BN, CW), lambda i: (0, 0, i, 0, 0)),
            pl.BlockSpec((NQ, BN, Q), lambda i: (0, i, 0)),
            pl.BlockSpec(((T + 1) * D, D), lambda i: (0, 0)),
            pl.BlockSpec((1, D), lambda i: (0, 0)),
        ],
        out_specs=pl.BlockSpec((1, D), lambda i: (0, 0)),
        out_shape=jax.ShapeDtypeStruct((1, D), jnp.float32),
    )(sums.reshape(NC, T, N, H), cnt.reshape(NC, T, GRID, BN, CW), enc4, W_agg,
      b_agg.reshape(1, D))
    return out.reshape(D)


def kernel(x_node_feature, edge_index, node_type, W_content, b_content, W_agg, b_agg):
    nt = node_type.astype(jnp.int32)
    src = edge_index[0].astype(jnp.int32)
    dst = edge_index[1].astype(jnp.int32)
    enc4, ench = _encode(x_node_feature, nt, W_content, b_content)
    sums, cnt = _segsum(ench, src, dst, nt)
    return _finish(sums, cnt, enc4, W_agg, b_agg)


# speedup vs baseline: 14.3464x; 1.0325x over previous
import jax
import jax.numpy as jnp
from jax import lax
from jax.experimental import pallas as pl
from jax.experimental.pallas import tpu as pltpu
from jax.experimental.pallas import tpu_sc as plsc

N = 10000
E = 320000
D = 128
T = 3
NQ = 4
Q = D // NQ
H = D // 2

NC = 2
NS = 16
K = 128
NCHUNK = E // K
MAXCH = 160
NW = 4
WCH = MAXCH // NW
SEGS = N * T
SEGS_PAD = SEGS + 16
ROWS_PER_TILE = 2000
NBUF = 8
CW = 16

BN = 1000
GRID = N // BN


def _encode_body(x_ref, nt_ref, w_ref, b_ref, out_ref, outh_ref):
    x = x_ref[...]
    nt = nt_ref[...].reshape(BN, 1)
    acc = jnp.zeros((BN, D), jnp.float32)
    for t in range(T):
        e = lax.dot_general(x, w_ref[t], (((1,), (0,)), ((), ())),
                            preferred_element_type=jnp.float32)
        e = e + b_ref[t][None, :]
        acc = acc + jnp.where(nt == t, e, 0.0)
    acc = jnp.where(acc >= 0.0, acc, 0.01 * acc)
    for q in range(NQ):
        out_ref[q] = acc[:, q * Q:(q + 1) * Q]
    for h in range(2):
        outh_ref[h] = acc[:, h * H:(h + 1) * H].astype(jnp.bfloat16)


def _encode(x, node_type, W_content, b_content):
    return pl.pallas_call(
        _encode_body,
        grid=(GRID,),
        in_specs=[
            pl.BlockSpec((BN, D), lambda i: (i, 0)),
            pl.BlockSpec((1, 1, BN), lambda i: (i, 0, 0)),
            pl.BlockSpec((T, D, D), lambda i: (0, 0, 0)),
            pl.BlockSpec((T, D), lambda i: (0, 0)),
        ],
        out_specs=[pl.BlockSpec((NQ, BN, Q), lambda i: (0, i, 0)),
                   pl.BlockSpec((2, BN, H), lambda i: (0, i, 0))],
        out_shape=[jax.ShapeDtypeStruct((NQ, N, Q), jnp.float32),
                   jax.ShapeDtypeStruct((2, N, H), jnp.bfloat16)],
    )(x, node_type.reshape(GRID, 1, BN), W_content, b_content)


def _segsum_body(enc_hbm, src_hbm, dst_hbm, nt_hbm, zrows_hbm, zcnt_hbm, ones_hbm,
                 sums_out, cnt_out,
                 sums_sp, cnt_sp, ntb, srcb, sidx, ones,
                 rows0, rows1, rows2, rows3, rows4, rows5, rows6, rows7,
                 gs0, gs1, gs2, gs3, gs4, gs5, gs6, gs7,
                 ss0, ss1, ss2, ss3, ss4, ss5, ss6, ss7,
                 cs0, cs1, cs2, cs3, cs4, cs5, cs6, cs7):
    rows = (rows0, rows1, rows2, rows3, rows4, rows5, rows6, rows7)
    gsem = (gs0, gs1, gs2, gs3, gs4, gs5, gs6, gs7)
    ssem = (ss0, ss1, ss2, ss3, ss4, ss5, ss6, ss7)
    csem = (cs0, cs1, cs2, cs3, cs4, cs5, cs6, cs7)
    c = lax.axis_index("c")
    s = lax.axis_index("s")
    ch0 = s * MAXCH
    nch = jnp.minimum(NCHUNK - ch0, MAXCH)
    r0 = s * ROWS_PER_TILE

    pltpu.sync_copy(nt_hbm, ntb)
    pltpu.sync_copy(ones_hbm, ones)

    @pl.when(s < NS - 1)
    def _():
        pltpu.sync_copy(zrows_hbm, sums_sp.at[pl.ds(r0, ROWS_PER_TILE)])
        pltpu.sync_copy(zcnt_hbm, cnt_sp.at[pl.ds(r0, ROWS_PER_TILE)])

    dummy = SEGS + lax.iota(jnp.int32, 16)
    plsc.subcore_barrier()

    for w in range(NW):
        pltpu.sync_copy(src_hbm.at[pl.ds(ch0 + w * WCH, WCH)], srcb)
        pltpu.sync_copy(dst_hbm.at[pl.ds(ch0 + w * WCH, WCH)], sidx)

        @plsc.parallel_loop(0, WCH, step=1, unroll=2)
        def _(k):
            real = w * WCH + k < nch
            for j in range(K // 16):
                sl = pl.ds(j * 16, 16)
                sv = srcb[k, sl]
                dv = sidx[k, sl]
                tv = plsc.load_gather(ntb, [sv])
                sidx[k, sl] = jnp.where(real, tv * N + dv, dummy)
                srcb[k, sl] = sv + c * N

        for b in range(NBUF):
            pltpu.async_copy(enc_hbm.at[srcb.at[b]], rows[b], gsem[b])

        def round_body(m, _):
            k = m * NBUF
            for b in range(NBUF):
                pltpu.make_async_copy(enc_hbm.at[srcb.at[k + b]],
                                      rows[b], gsem[b]).wait()
                pltpu.async_copy(rows[b], sums_sp.at[sidx.at[k + b]], ssem[b],
                                 add=True)

                @pl.when(c == (b % 2))
                def _():
                    pltpu.async_copy(ones, cnt_sp.at[sidx.at[k + b]], csem[b],
                                     add=True)
            for b in range(NBUF):
                pltpu.make_async_copy(rows[b], sums_sp.at[sidx.at[k + b]],
                                      ssem[b]).wait()

                @pl.when(c == (b % 2))
                def _():
                    pltpu.make_async_copy(ones, cnt_sp.at[sidx.at[k + b]],
                                          csem[b]).wait()

                @pl.when(k + NBUF + b < WCH)
                def _():
                    pltpu.async_copy(enc_hbm.at[srcb.at[k + NBUF + b]],
                                     rows[b], gsem[b])
            return 0

        lax.fori_loop(0, WCH // NBUF, round_body, 0)

    plsc.subcore_barrier()

    @pl.when(s < NS - 1)
    def _():
        pltpu.sync_copy(sums_sp.at[pl.ds(r0, ROWS_PER_TILE)],
                        sums_out.at[pl.ds(c * SEGS + r0, ROWS_PER_TILE)])
        pltpu.sync_copy(cnt_sp.at[pl.ds(r0, ROWS_PER_TILE)],
                        cnt_out.at[pl.ds(c * SEGS + r0, ROWS_PER_TILE)])


def _segsum(ench, src, dst, node_type):
    npad = NS * MAXCH * K - E
    srcp = jnp.concatenate([src, jnp.zeros((npad,), jnp.int32)]).reshape(NS * MAXCH, K)
    dstp = jnp.concatenate([dst, jnp.zeros((npad,), jnp.int32)]).reshape(NS * MAXCH, K)
    zrows = jnp.zeros((ROWS_PER_TILE, H), jnp.bfloat16)
    zcnt = jnp.zeros((ROWS_PER_TILE, CW), jnp.bfloat16)
    ones = jnp.ones((K, CW), jnp.bfloat16)
    mesh = plsc.VectorSubcoreMesh(core_axis_name="c", subcore_axis_name="s",
                                  num_cores=NC, num_subcores=NS)
    f = pl.kernel(
        _segsum_body,
        out_type=(jax.ShapeDtypeStruct((NC * SEGS, H), jnp.bfloat16),
                  jax.ShapeDtypeStruct((NC * SEGS, CW), jnp.bfloat16)),
        mesh=mesh,
        compiler_params=pltpu.CompilerParams(needs_layout_passes=False,
                                             use_tc_tiling_on_sc=False),
        scratch_types=(
            [pltpu.VMEM_SHARED((SEGS_PAD, H), jnp.bfloat16),
             pltpu.VMEM_SHARED((SEGS_PAD, CW), jnp.bfloat16),
             pltpu.VMEM((N,), jnp.int32),
             pltpu.VMEM((WCH, K), jnp.int32),
             pltpu.VMEM((WCH, K), jnp.int32),
             pltpu.VMEM((K, CW), jnp.bfloat16)]
            + [pltpu.VMEM((K, H), jnp.bfloat16)] * NBUF
            + [pltpu.SemaphoreType.DMA] * (3 * NBUF)
        ),
    )
    return f(ench.reshape(2 * N, H), srcp, dstp, node_type, zrows, zcnt, ones)


def _finish_body(sums_ref, cnt_ref, enc_ref, w_ref, b_ref, out_ref):
    i = pl.program_id(0)
    cnt = (cnt_ref[0, :, 0, :, 0].astype(jnp.float32)
           + cnt_ref[1, :, 0, :, 0].astype(jnp.float32))
    pieces = []
    for t in range(T):
        denom = jnp.maximum(cnt[t], 1.0)[:, None]
        for h in range(2):
            pieces.append(sums_ref[h, t].astype(jnp.float32) / denom)
    for q in range(NQ):
        pieces.append(enc_ref[q])
    het = jnp.concatenate(pieces, axis=1)
    z = lax.dot_general(het, w_ref[...], (((1,), (0,)), ((), ())),
                        preferred_element_type=jnp.float32)
    z = z + b_ref[...]
    emb = 1.0 / (1.0 + jnp.exp(-z))
    part = jnp.sum(emb, axis=0, keepdims=True)

    @pl.when(i == 0)
    def _():
        out_ref[...] = jnp.zeros((1, D), jnp.float32)

    out_ref[...] += part

    @pl.when(i == GRID - 1)
    def _():
        out_ref[...] = out_ref[...] * (1.0 / N)


def _finish(sums, cnt, enc4, W_agg, b_agg):
    out = pl.pallas_call(
        _finish_body,
        grid=(GRID,),
        in_specs=[
            pl.BlockSpec((NC, T, BN, H), lambda i: (0, 0, i, 0)),
            pl.BlockSpec((NC, T, 1, BN, CW), lambda i: (0, 0, i, 0, 0)),
            pl.BlockSpec((NQ, BN, Q), lambda i: (0, i, 0)),
            pl.BlockSpec(((T + 1) * D, D), lambda i: (0, 0)),
            pl.BlockSpec((1, D), lambda i: (0, 0)),
        ],
        out_specs=pl.BlockSpec((1, D), lambda i: (0, 0)),
        out_shape=jax.ShapeDtypeStruct((1, D), jnp.float32),
    )(sums.reshape(NC, T, N, H), cnt.reshape(NC, T, GRID, BN, CW), enc4, W_agg,
      b_agg.reshape(1, D))
    return out.reshape(D)


def kernel(x_node_feature, edge_index, node_type, W_content, b_content, W_agg, b_agg):
    nt = node_type.astype(jnp.int32)
    src = edge_index[0].astype(jnp.int32)
    dst = edge_index[1].astype(jnp.int32)
    enc4, ench = _encode(x_node_feature, nt, W_content, b_content)
    sums, cnt = _segsum(ench, src, dst, nt)
    return _finish(sums, cnt, enc4, W_agg, b_agg)


# speedup vs baseline: 14.4841x; 1.0096x over previous
import jax
import jax.numpy as jnp
from jax import lax
from jax.experimental import pallas as pl
from jax.experimental.pallas import tpu as pltpu
from jax.experimental.pallas import tpu_sc as plsc

N = 10000
E = 320000
D = 128
T = 3
NQ = 4
Q = D // NQ
H = D // 2

NC = 2
NS = 16
K = 128
NCHUNK = E // K
MAXCH = 160
NW = 4
WCH = MAXCH // NW
SEGS = N * T
SEGS_PAD = SEGS + 16
ROWS_PER_TILE = 2000
NBUF = 8
CW = 16

BN = 1000
GRID = N // BN


def _encode_body(x_ref, nt_ref, w_ref, b_ref, out_ref, outh_ref):
    x = x_ref[...]
    nt = nt_ref[...].reshape(BN, 1)
    acc = jnp.zeros((BN, D), jnp.float32)
    for t in range(T):
        e = lax.dot_general(x, w_ref[t], (((1,), (0,)), ((), ())),
                            preferred_element_type=jnp.float32)
        e = e + b_ref[t][None, :]
        acc = acc + jnp.where(nt == t, e, 0.0)
    acc = jnp.where(acc >= 0.0, acc, 0.01 * acc)
    for q in range(NQ):
        out_ref[q] = acc[:, q * Q:(q + 1) * Q]
    for h in range(2):
        outh_ref[h] = acc[:, h * H:(h + 1) * H].astype(jnp.bfloat16)


def _encode(x, node_type, W_content, b_content):
    return pl.pallas_call(
        _encode_body,
        grid=(GRID,),
        in_specs=[
            pl.BlockSpec((BN, D), lambda i: (i, 0)),
            pl.BlockSpec((1, 1, BN), lambda i: (i, 0, 0)),
            pl.BlockSpec((T, D, D), lambda i: (0, 0, 0)),
            pl.BlockSpec((T, D), lambda i: (0, 0)),
        ],
        out_specs=[pl.BlockSpec((NQ, BN, Q), lambda i: (0, i, 0)),
                   pl.BlockSpec((2, BN, H), lambda i: (0, i, 0))],
        out_shape=[jax.ShapeDtypeStruct((NQ, N, Q), jnp.float32),
                   jax.ShapeDtypeStruct((2, N, H), jnp.bfloat16)],
    )(x, node_type.reshape(GRID, 1, BN), W_content, b_content)


def _segsum_body(enc_hbm, ei_hbm, nt_hbm, zrows_hbm, zcnt_hbm, ones_hbm,
                 sums_out, cnt_out,
                 sums_sp, cnt_sp, ntb, srcb, sidx, ones,
                 rows0, rows1, rows2, rows3, rows4, rows5, rows6, rows7,
                 gs0, gs1, gs2, gs3, gs4, gs5, gs6, gs7,
                 ss0, ss1, ss2, ss3, ss4, ss5, ss6, ss7,
                 cs0, cs1, cs2, cs3, cs4, cs5, cs6, cs7):
    rows = (rows0, rows1, rows2, rows3, rows4, rows5, rows6, rows7)
    gsem = (gs0, gs1, gs2, gs3, gs4, gs5, gs6, gs7)
    ssem = (ss0, ss1, ss2, ss3, ss4, ss5, ss6, ss7)
    csem = (cs0, cs1, cs2, cs3, cs4, cs5, cs6, cs7)
    c = lax.axis_index("c")
    s = lax.axis_index("s")
    ch0 = s * MAXCH
    nch = jnp.minimum(NCHUNK - ch0, MAXCH)
    r0 = s * ROWS_PER_TILE

    pltpu.sync_copy(nt_hbm, ntb)
    pltpu.sync_copy(ones_hbm, ones)

    @pl.when(s < NS - 1)
    def _():
        pltpu.sync_copy(zrows_hbm, sums_sp.at[pl.ds(r0, ROWS_PER_TILE)])
        pltpu.sync_copy(zcnt_hbm, cnt_sp.at[pl.ds(r0, ROWS_PER_TILE)])

    dummy = SEGS + lax.iota(jnp.int32, 16)
    plsc.subcore_barrier()

    for w in range(NW):
        pltpu.sync_copy(ei_hbm.at[0].at[pl.ds(ch0 + w * WCH, WCH)], srcb)
        pltpu.sync_copy(ei_hbm.at[1].at[pl.ds(ch0 + w * WCH, WCH)], sidx)

        @plsc.parallel_loop(0, WCH, step=1, unroll=2)
        def _(k):
            real = w * WCH + k < nch
            for j in range(K // 16):
                sl = pl.ds(j * 16, 16)
                sv = srcb[k, sl]
                dv = sidx[k, sl]
                tv = plsc.load_gather(ntb, [sv])
                sidx[k, sl] = jnp.where(real, tv * N + dv, dummy)
                srcb[k, sl] = sv + c * N

        for b in range(NBUF):
            pltpu.async_copy(enc_hbm.at[srcb.at[b]], rows[b], gsem[b])

        def round_body(m, _):
            k = m * NBUF
            for b in range(NBUF):
                pltpu.make_async_copy(enc_hbm.at[srcb.at[k + b]],
                                      rows[b], gsem[b]).wait()
                pltpu.async_copy(rows[b], sums_sp.at[sidx.at[k + b]], ssem[b],
                                 add=True)

                @pl.when(c == (b % 2))
                def _():
                    pltpu.async_copy(ones, cnt_sp.at[sidx.at[k + b]], csem[b],
                                     add=True)
            for b in range(NBUF):
                pltpu.make_async_copy(rows[b], sums_sp.at[sidx.at[k + b]],
                                      ssem[b]).wait()

                @pl.when(k + NBUF + b < WCH)
                def _():
                    pltpu.async_copy(enc_hbm.at[srcb.at[k + NBUF + b]],
                                     rows[b], gsem[b])

                @pl.when(c == (b % 2))
                def _():
                    pltpu.make_async_copy(ones, cnt_sp.at[sidx.at[k + b]],
                                          csem[b]).wait()
            return 0

        lax.fori_loop(0, WCH // NBUF, round_body, 0)

    plsc.subcore_barrier()

    tt = r0 // N
    n0 = r0 - tt * N

    @pl.when(s < NS - 1)
    def _():
        pltpu.sync_copy(sums_sp.at[pl.ds(r0, ROWS_PER_TILE)],
                        sums_out.at[c].at[tt].at[pl.ds(n0, ROWS_PER_TILE)])
        pltpu.sync_copy(cnt_sp.at[pl.ds(r0, ROWS_PER_TILE)],
                        cnt_out.at[c].at[tt].at[pl.ds(n0, ROWS_PER_TILE)])


def _segsum(ench, edge_index, node_type):
    npad = NS * MAXCH * K - E
    eip = jnp.pad(edge_index.astype(jnp.int32),
                  ((0, 0), (0, npad))).reshape(2, NS * MAXCH, K)
    zrows = jnp.zeros((ROWS_PER_TILE, H), jnp.bfloat16)
    zcnt = jnp.zeros((ROWS_PER_TILE, CW), jnp.bfloat16)
    ones = jnp.ones((K, CW), jnp.bfloat16)
    mesh = plsc.VectorSubcoreMesh(core_axis_name="c", subcore_axis_name="s",
                                  num_cores=NC, num_subcores=NS)
    f = pl.kernel(
        _segsum_body,
        out_type=(jax.ShapeDtypeStruct((NC, T, N, H), jnp.bfloat16),
                  jax.ShapeDtypeStruct((NC, T, N, CW), jnp.bfloat16)),
        mesh=mesh,
        compiler_params=pltpu.CompilerParams(needs_layout_passes=False,
                                             use_tc_tiling_on_sc=False),
        scratch_types=(
            [pltpu.VMEM_SHARED((SEGS_PAD, H), jnp.bfloat16),
             pltpu.VMEM_SHARED((SEGS_PAD, CW), jnp.bfloat16),
             pltpu.VMEM((N,), jnp.int32),
             pltpu.VMEM((WCH, K), jnp.int32),
             pltpu.VMEM((WCH, K), jnp.int32),
             pltpu.VMEM((K, CW), jnp.bfloat16)]
            + [pltpu.VMEM((K, H), jnp.bfloat16)] * NBUF
            + [pltpu.SemaphoreType.DMA] * (3 * NBUF)
        ),
    )
    return f(ench.reshape(2 * N, H), eip, node_type, zrows, zcnt, ones)


def _finish_body(sums_ref, cnt_ref, enc_ref, w_ref, b_ref, out_ref):
    i = pl.program_id(0)
    cnt = (cnt_ref[0, :, :, 0].astype(jnp.float32)
           + cnt_ref[1, :, :, 0].astype(jnp.float32))
    pieces = []
    for t in range(T):
        denom = jnp.maximum(cnt[t], 1.0)[:, None]
        for h in range(2):
            pieces.append(sums_ref[h, t].astype(jnp.float32) / denom)
    for q in range(NQ):
        pieces.append(enc_ref[q])
    het = jnp.concatenate(pieces, axis=1)
    z = lax.dot_general(het, w_ref[...], (((1,), (0,)), ((), ())),
                        preferred_element_type=jnp.float32)
    z = z + b_ref[...]
    emb = 1.0 / (1.0 + jnp.exp(-z))
    part = jnp.sum(emb, axis=0, keepdims=True)

    @pl.when(i == 0)
    def _():
        out_ref[...] = jnp.zeros((1, D), jnp.float32)

    out_ref[...] += part

    @pl.when(i == GRID - 1)
    def _():
        out_ref[...] = out_ref[...] * (1.0 / N)


def _finish(sums, cnt, enc4, W_agg, b_agg):
    out = pl.pallas_call(
        _finish_body,
        grid=(GRID,),
        in_specs=[
            pl.BlockSpec((NC, T, BN, H), lambda i: (0, 0, i, 0)),
            pl.BlockSpec((NC, T, BN, CW), lambda i: (0, 0, i, 0)),
            pl.BlockSpec((NQ, BN, Q), lambda i: (0, i, 0)),
            pl.BlockSpec(((T + 1) * D, D), lambda i: (0, 0)),
            pl.BlockSpec((1, D), lambda i: (0, 0)),
        ],
        out_specs=pl.BlockSpec((1, D), lambda i: (0, 0)),
        out_shape=jax.ShapeDtypeStruct((1, D), jnp.float32),
    )(sums, cnt, enc4, W_agg, b_agg.reshape(1, D))
    return out.reshape(D)


def kernel(x_node_feature, edge_index, node_type, W_content, b_content, W_agg, b_agg):
    nt = node_type.astype(jnp.int32)
    enc4, ench = _encode(x_node_feature, nt, W_content, b_content)
    sums, cnt = _segsum(ench, edge_index, nt)
    return _finish(sums, cnt, enc4, W_agg, b_agg)


# speedup vs baseline: 14.4951x; 1.0008x over previous
import jax
import jax.numpy as jnp
from jax import lax
from jax.experimental import pallas as pl
from jax.experimental.pallas import tpu as pltpu
from jax.experimental.pallas import tpu_sc as plsc

N = 10000
E = 320000
D = 128
T = 3
NQ = 4
Q = D // NQ
H = D // 2

NC = 2
NS = 16
K = 128
NCHUNK = E // K
MAXCH = 160
NW = 4
WCH = MAXCH // NW
SEGS = N * T
SEGS_PAD = SEGS + 16
ROWS_PER_TILE = 2000
NBUF = 8
CW = 16

BN = 1000
GRID = N // BN


def _encode_body(x_ref, nt_ref, w_ref, b_ref, out_ref, outh_ref):
    x = x_ref[...]
    nt = nt_ref[...].reshape(BN, 1)
    acc = jnp.zeros((BN, D), jnp.float32)
    for t in range(T):
        e = lax.dot_general(x, w_ref[t], (((1,), (0,)), ((), ())),
                            preferred_element_type=jnp.float32)
        e = e + b_ref[t][None, :]
        acc = acc + jnp.where(nt == t, e, 0.0)
    acc = jnp.where(acc >= 0.0, acc, 0.01 * acc)
    for q in range(NQ):
        out_ref[q] = acc[:, q * Q:(q + 1) * Q]
    for h in range(2):
        outh_ref[h] = acc[:, h * H:(h + 1) * H].astype(jnp.bfloat16)


def _encode(x, node_type, W_content, b_content):
    return pl.pallas_call(
        _encode_body,
        grid=(GRID,),
        in_specs=[
            pl.BlockSpec((BN, D), lambda i: (i, 0)),
            pl.BlockSpec((1, 1, BN), lambda i: (i, 0, 0)),
            pl.BlockSpec((T, D, D), lambda i: (0, 0, 0)),
            pl.BlockSpec((T, D), lambda i: (0, 0)),
        ],
        out_specs=[pl.BlockSpec((NQ, BN, Q), lambda i: (0, i, 0)),
                   pl.BlockSpec((2, BN, H), lambda i: (0, i, 0))],
        out_shape=[jax.ShapeDtypeStruct((NQ, N, Q), jnp.float32),
                   jax.ShapeDtypeStruct((2, N, H), jnp.bfloat16)],
    )(x, node_type.reshape(GRID, 1, BN), W_content, b_content)


def _segsum_body(enc_hbm, ei_hbm, nt_hbm, zrows_hbm, zcnt_hbm, ones_hbm,
                 sums_out, cnt_out,
                 sums_sp, cnt_sp, ntb, srcb, sidx, ones,
                 rows0, rows1, rows2, rows3, rows4, rows5, rows6, rows7,
                 gs0, gs1, gs2, gs3, gs4, gs5, gs6, gs7,
                 ss0, ss1, ss2, ss3, ss4, ss5, ss6, ss7,
                 cs0, cs1, cs2, cs3, cs4, cs5, cs6, cs7):
    rows = (rows0, rows1, rows2, rows3, rows4, rows5, rows6, rows7)
    gsem = (gs0, gs1, gs2, gs3, gs4, gs5, gs6, gs7)
    ssem = (ss0, ss1, ss2, ss3, ss4, ss5, ss6, ss7)
    csem = (cs0, cs1, cs2, cs3, cs4, cs5, cs6, cs7)
    c = lax.axis_index("c")
    s = lax.axis_index("s")
    ch0 = s * MAXCH
    nch = jnp.minimum(NCHUNK - ch0, MAXCH)
    r0 = s * ROWS_PER_TILE

    pltpu.sync_copy(nt_hbm, ntb)
    pltpu.sync_copy(ones_hbm, ones)

    @pl.when(s < NS - 1)
    def _():
        pltpu.sync_copy(zrows_hbm, sums_sp.at[pl.ds(r0, ROWS_PER_TILE)])
        pltpu.sync_copy(zcnt_hbm, cnt_sp.at[pl.ds(r0, ROWS_PER_TILE)])

    dummy = SEGS + lax.iota(jnp.int32, 16)
    plsc.subcore_barrier()

    for w in range(NW):
        pltpu.sync_copy(ei_hbm.at[0].at[pl.ds(ch0 + w * WCH, WCH)], srcb)
        pltpu.sync_copy(ei_hbm.at[1].at[pl.ds(ch0 + w * WCH, WCH)], sidx)

        @plsc.parallel_loop(0, WCH, step=1, unroll=2)
        def _(k):
            real = w * WCH + k < nch
            for j in range(K // 16):
                sl = pl.ds(j * 16, 16)
                sv = srcb[k, sl]
                dv = sidx[k, sl]
                tv = plsc.load_gather(ntb, [sv])
                sidx[k, sl] = jnp.where(real, tv * N + dv, dummy)

        enc_c = enc_hbm.at[c]
        for b in range(NBUF):
            pltpu.async_copy(enc_c.at[srcb.at[b]], rows[b], gsem[b])

        def round_body(m, _):
            k = m * NBUF
            for b in range(NBUF):
                pltpu.make_async_copy(enc_c.at[srcb.at[k + b]],
                                      rows[b], gsem[b]).wait()
                pltpu.async_copy(rows[b], sums_sp.at[sidx.at[k + b]], ssem[b],
                                 add=True)

                @pl.when(c == (b % 2))
                def _():
                    pltpu.async_copy(ones, cnt_sp.at[sidx.at[k + b]], csem[b],
                                     add=True)
            for b in range(NBUF):
                pltpu.make_async_copy(rows[b], sums_sp.at[sidx.at[k + b]],
                                      ssem[b]).wait()

                @pl.when(k + NBUF + b < WCH)
                def _():
                    pltpu.async_copy(enc_c.at[srcb.at[k + NBUF + b]],
                                     rows[b], gsem[b])

                @pl.when(c == (b % 2))
                def _():
                    pltpu.make_async_copy(ones, cnt_sp.at[sidx.at[k + b]],
                                          csem[b]).wait()
            return 0

        lax.fori_loop(0, WCH // NBUF, round_body, 0)

    plsc.subcore_barrier()

    tt = r0 // N
    n0 = r0 - tt * N

    @pl.when(s < NS - 1)
    def _():
        pltpu.sync_copy(sums_sp.at[pl.ds(r0, ROWS_PER_TILE)],
                        sums_out.at[c].at[tt].at[pl.ds(n0, ROWS_PER_TILE)])
        pltpu.sync_copy(cnt_sp.at[pl.ds(r0, ROWS_PER_TILE)],
                        cnt_out.at[c].at[tt].at[pl.ds(n0, ROWS_PER_TILE)])


def _segsum(ench, edge_index, node_type):
    npad = NS * MAXCH * K - E
    eip = jnp.pad(edge_index.astype(jnp.int32),
                  ((0, 0), (0, npad))).reshape(2, NS * MAXCH, K)
    zrows = jnp.zeros((ROWS_PER_TILE, H), jnp.bfloat16)
    zcnt = jnp.zeros((ROWS_PER_TILE, CW), jnp.bfloat16)
    ones = jnp.ones((K, CW), jnp.bfloat16)
    mesh = plsc.VectorSubcoreMesh(core_axis_name="c", subcore_axis_name="s",
                                  num_cores=NC, num_subcores=NS)
    f = pl.kernel(
        _segsum_body,
        out_type=(jax.ShapeDtypeStruct((NC, T, N, H), jnp.bfloat16),
                  jax.ShapeDtypeStruct((NC, T, N, CW), jnp.bfloat16)),
        mesh=mesh,
        compiler_params=pltpu.CompilerParams(needs_layout_passes=False,
                                             use_tc_tiling_on_sc=False),
        scratch_types=(
            [pltpu.VMEM_SHARED((SEGS_PAD, H), jnp.bfloat16),
             pltpu.VMEM_SHARED((SEGS_PAD, CW), jnp.bfloat16),
             pltpu.VMEM((N,), jnp.int32),
             pltpu.VMEM((WCH, K), jnp.int32),
             pltpu.VMEM((WCH, K), jnp.int32),
             pltpu.VMEM((K, CW), jnp.bfloat16)]
            + [pltpu.VMEM((K, H), jnp.bfloat16)] * NBUF
            + [pltpu.SemaphoreType.DMA] * (3 * NBUF)
        ),
    )
    return f(ench, eip, node_type, zrows, zcnt, ones)


def _finish_body(sums_ref, cnt_ref, enc_ref, w_ref, b_ref, out_ref):
    i = pl.program_id(0)
    cnt = (cnt_ref[0, :, :, 0].astype(jnp.float32)
           + cnt_ref[1, :, :, 0].astype(jnp.float32))
    pieces = []
    for t in range(T):
        denom = jnp.maximum(cnt[t], 1.0)[:, None]
        for h in range(2):
            pieces.append(sums_ref[h, t].astype(jnp.float32) / denom)
    for q in range(NQ):
        pieces.append(enc_ref[q])
    het = jnp.concatenate(pieces, axis=1)
    z = lax.dot_general(het, w_ref[...], (((1,), (0,)), ((), ())),
                        preferred_element_type=jnp.float32)
    z = z + b_ref[...]
    emb = 1.0 / (1.0 + jnp.exp(-z))
    part = jnp.sum(emb, axis=0)

    @pl.when(i == 0)
    def _():
        out_ref[...] = jnp.zeros((D,), jnp.float32)

    out_ref[...] += part

    @pl.when(i == GRID - 1)
    def _():
        out_ref[...] = out_ref[...] * (1.0 / N)


def _finish(sums, cnt, enc4, W_agg, b_agg):
    out = pl.pallas_call(
        _finish_body,
        grid=(GRID,),
        in_specs=[
            pl.BlockSpec((NC, T, BN, H), lambda i: (0, 0, i, 0)),
            pl.BlockSpec((NC, T, BN, CW), lambda i: (0, 0, i, 0)),
            pl.BlockSpec((NQ, BN, Q), lambda i: (0, i, 0)),
            pl.BlockSpec(((T + 1) * D, D), lambda i: (0, 0)),
            pl.BlockSpec((1, D), lambda i: (0, 0)),
        ],
        out_specs=pl.BlockSpec((D,), lambda i: (0,)),
        out_shape=jax.ShapeDtypeStruct((D,), jnp.float32),
    )(sums, cnt, enc4, W_agg, b_agg.reshape(1, D))
    return out


def kernel(x_node_feature, edge_index, node_type, W_content, b_content, W_agg, b_agg):
    nt = node_type.astype(jnp.int32)
    enc4, ench = _encode(x_node_feature, nt, W_content, b_content)
    sums, cnt = _segsum(ench, edge_index, nt)
    return _finish(sums, cnt, enc4, W_agg, b_agg)
